# initial kernel scaffold (unmeasured)
import functools

import jax
import jax.numpy as jnp
from jax import lax
from jax.experimental import pallas as pl
from jax.experimental.pallas import tpu as pltpu

N_DEV = 4
M_OUT = 512
N_COLS = 8192


def kernel(x, dy):
    p = jnp.dot(x.T, dy, preferred_element_type=jnp.float32)

    def body(p_ref, out_ref, comm_ref, snd_ref, tmp_ref,
             send_sems, recv_sems, local_sem):
        my_x = lax.axis_index("x")
        my_y = lax.axis_index("y")
        my_z = lax.axis_index("z")
        left = (my_y - 1) % N_DEV
        right = (my_y + 1) % N_DEV

        def load_chunk(d, dst):
            cp = pltpu.make_async_copy(
                p_ref.at[pl.ds(d * M_OUT, M_OUT), :], dst, local_sem)
            cp.start()
            cp.wait()

        load_chunk((my_y - 1) % N_DEV, snd_ref)

        barrier_sem = pltpu.get_barrier_semaphore()
        for nbr in (left, right):
            pl.semaphore_signal(
                barrier_sem, inc=1,
                device_id=(my_x, nbr, my_z),
                device_id_type=pl.DeviceIdType.MESH)
        pl.semaphore_wait(barrier_sem, 2)

        for h in range(N_DEV - 1):
            src = snd_ref if h == 0 else comm_ref.at[h - 1]
            rdma = pltpu.make_async_remote_copy(
                src_ref=src,
                dst_ref=comm_ref.at[h],
                send_sem=send_sems.at[h],
                recv_sem=recv_sems.at[h],
                device_id=(my_x, right, my_z),
                device_id_type=pl.DeviceIdType.MESH)
            rdma.start()
            rdma.wait()

            load_chunk((my_y - 2 - h) % N_DEV, tmp_ref)
            if h < N_DEV - 2:
                comm_ref[h] = comm_ref[h] + tmp_ref[:, :]
            else:
                out_ref[:, :] = comm_ref[h] + tmp_ref[:, :]

        @functools.partial(
            pl.run_scoped, second_barrier=pltpu.SemaphoreType.REGULAR)
        def _(second_barrier):
            for nbr in (left, right):
                pl.semaphore_signal(
                    second_barrier, inc=1,
                    device_id=(my_x, nbr, my_z),
                    device_id_type=pl.DeviceIdType.MESH)
            pl.semaphore_wait(second_barrier, 2)

    out_shape = jax.ShapeDtypeStruct((M_OUT, N_COLS), jnp.float32)
    return pl.pallas_call(
        body,
        out_shape=out_shape,
        in_specs=[pl.BlockSpec(memory_space=pltpu.ANY)],
        out_specs=pl.BlockSpec(memory_space=pltpu.VMEM),
        scratch_shapes=[
            pltpu.VMEM((N_DEV - 1, M_OUT, N_COLS), jnp.float32),
            pltpu.VMEM((M_OUT, N_COLS), jnp.float32),
            pltpu.VMEM((M_OUT, N_COLS), jnp.float32),
            pltpu.SemaphoreType.DMA((N_DEV - 1,)),
            pltpu.SemaphoreType.DMA((N_DEV - 1,)),
            pltpu.SemaphoreType.DMA,
        ],
        compiler_params=pltpu.CompilerParams(collective_id=0),
    )(p)


# baseline (device time: 690329 ns/iter reference)
import functools

import jax
import jax.numpy as jnp
from jax import lax
from jax.experimental import pallas as pl
from jax.experimental.pallas import tpu as pltpu

N_DEV = 4
M_OUT = 512
N_COLS = 8192
HALF = N_COLS // 2


def kernel(x, dy):
    p = jnp.dot(x.T, dy, preferred_element_type=jnp.float32)

    def body(p_ref, out_ref, comm_ref, tmp_ref, send_sems, recv_sems,
             local_sem, credit_sem):
        my_x = lax.axis_index("x")
        my_y = lax.axis_index("y")
        my_z = lax.axis_index("z")
        left = (my_y - 1) % N_DEV
        right = (my_y + 1) % N_DEV

        barrier_sem = pltpu.get_barrier_semaphore()
        for nbr in (left, right):
            pl.semaphore_signal(
                barrier_sem, inc=1,
                device_id=(my_x, nbr, my_z),
                device_id_type=pl.DeviceIdType.MESH)
        pl.semaphore_wait(barrier_sem, 2)

        def add_local(d, slot, final):
            for ci in range(2):
                cp = pltpu.make_async_copy(
                    p_ref.at[pl.ds(d * M_OUT, M_OUT),
                             pl.ds(ci * HALF, HALF)],
                    tmp_ref, local_sem)
                cp.start()
                cp.wait()
                s = comm_ref[slot][:, ci * HALF:(ci + 1) * HALF] + tmp_ref[:, :]
                if final:
                    out_ref[:, ci * HALF:(ci + 1) * HALF] = s
                else:
                    comm_ref[slot, :, ci * HALF:(ci + 1) * HALF] = s

        for h in range(N_DEV - 1):
            slot = h % 2
            if h == 0:
                src = p_ref.at[pl.ds(left * M_OUT, M_OUT), :]
            else:
                src = comm_ref.at[(h - 1) % 2]
            if h == 2:
                pl.semaphore_wait(credit_sem, 1)
            rdma = pltpu.make_async_remote_copy(
                src_ref=src,
                dst_ref=comm_ref.at[slot],
                send_sem=send_sems.at[slot],
                recv_sem=recv_sems.at[slot],
                device_id=(my_x, right, my_z),
                device_id_type=pl.DeviceIdType.MESH)
            rdma.start()
            rdma.wait()
            if h == 1:
                pl.semaphore_signal(
                    credit_sem, inc=1,
                    device_id=(my_x, left, my_z),
                    device_id_type=pl.DeviceIdType.MESH)
            add_local((my_y - 2 - h) % N_DEV, slot, final=(h == N_DEV - 2))

        @functools.partial(
            pl.run_scoped, second_barrier=pltpu.SemaphoreType.REGULAR)
        def _(second_barrier):
            for nbr in (left, right):
                pl.semaphore_signal(
                    second_barrier, inc=1,
                    device_id=(my_x, nbr, my_z),
                    device_id_type=pl.DeviceIdType.MESH)
            pl.semaphore_wait(second_barrier, 2)

    out_shape = jax.ShapeDtypeStruct((M_OUT, N_COLS), jnp.float32)
    return pl.pallas_call(
        body,
        out_shape=out_shape,
        in_specs=[pl.BlockSpec(memory_space=pl.ANY)],
        out_specs=pl.BlockSpec(memory_space=pltpu.VMEM),
        scratch_shapes=[
            pltpu.VMEM((2, M_OUT, N_COLS), jnp.float32),
            pltpu.VMEM((M_OUT, HALF), jnp.float32),
            pltpu.SemaphoreType.DMA((2,)),
            pltpu.SemaphoreType.DMA((2,)),
            pltpu.SemaphoreType.DMA,
            pltpu.SemaphoreType.REGULAR,
        ],
        compiler_params=pltpu.CompilerParams(
            collective_id=0, vmem_limit_bytes=63 * 1024 * 1024),
    )(p)


# device time: 289840 ns/iter; 2.3818x vs baseline; 2.3818x over previous
import functools

import jax
import jax.numpy as jnp
from jax import lax
from jax.experimental import pallas as pl
from jax.experimental.pallas import tpu as pltpu

N_Y = 4
N_XZ = 8
M_OUT = 512
N_COLS = 8192
W = N_COLS // N_XZ


def _ring_pos(my_x, my_z):
    return lax.select(my_x == 0, my_z, 7 - my_z)


def _pos_to_xz(pos):
    tx = pos // 4
    tz = lax.select(tx == 0, pos, 7 - pos)
    return tx, tz


def kernel(x, dy):
    my_x = lax.axis_index("x")
    my_z = lax.axis_index("z")
    pos = _ring_pos(my_x, my_z)
    c0 = pos * W
    dy_slice = lax.dynamic_slice_in_dim(dy, c0, W, axis=1)
    q = jnp.dot(x.T, dy_slice, preferred_element_type=jnp.float32)

    def body(q_ref, out_ref, comm_a, comm_b,
             a_send, a_recv, b_send, b_recv):
        my_x = lax.axis_index("x")
        my_y = lax.axis_index("y")
        my_z = lax.axis_index("z")
        pos = _ring_pos(my_x, my_z)
        left = (my_y - 1) % N_Y
        right = (my_y + 1) % N_Y
        nxt_x, nxt_z = _pos_to_xz((pos + 1) % N_XZ)
        prv_x, prv_z = _pos_to_xz((pos - 1) % N_XZ)

        peers = [(my_x, left, my_z), (my_x, right, my_z),
                 (nxt_x, my_y, nxt_z), (prv_x, my_y, prv_z)]
        barrier_sem = pltpu.get_barrier_semaphore()
        for p in peers:
            pl.semaphore_signal(
                barrier_sem, inc=1, device_id=p,
                device_id_type=pl.DeviceIdType.MESH)
        pl.semaphore_wait(barrier_sem, 4)

        for h in range(N_Y - 1):
            src = (q_ref.at[pl.ds(left * M_OUT, M_OUT), :] if h == 0
                   else comm_a.at[h - 1])
            rdma = pltpu.make_async_remote_copy(
                src_ref=src,
                dst_ref=comm_a.at[h],
                send_sem=a_send.at[h],
                recv_sem=a_recv.at[h],
                device_id=(my_x, right, my_z),
                device_id_type=pl.DeviceIdType.MESH)
            rdma.start()
            rdma.wait()
            d = (my_y - 2 - h) % N_Y
            comm_a[h] = comm_a[h] + q_ref[pl.ds(d * M_OUT, M_OUT), :]
        out_ref[:, pl.ds(pos * W, W)] = comm_a[N_Y - 2]

        for h in range(N_XZ - 1):
            src = comm_a.at[N_Y - 2] if h == 0 else comm_b.at[h - 1]
            rdma = pltpu.make_async_remote_copy(
                src_ref=src,
                dst_ref=comm_b.at[h],
                send_sem=b_send.at[h],
                recv_sem=b_recv.at[h],
                device_id=(nxt_x, my_y, nxt_z),
                device_id_type=pl.DeviceIdType.MESH)
            rdma.start()
            rdma.wait()
            origin = (pos - 1 - h) % N_XZ
            out_ref[:, pl.ds(origin * W, W)] = comm_b[h]

        @functools.partial(
            pl.run_scoped, second_barrier=pltpu.SemaphoreType.REGULAR)
        def _(second_barrier):
            for p in peers:
                pl.semaphore_signal(
                    second_barrier, inc=1, device_id=p,
                    device_id_type=pl.DeviceIdType.MESH)
            pl.semaphore_wait(second_barrier, 4)

    out_shape = jax.ShapeDtypeStruct((M_OUT, N_COLS), jnp.float32)
    return pl.pallas_call(
        body,
        out_shape=out_shape,
        in_specs=[pl.BlockSpec(memory_space=pltpu.VMEM)],
        out_specs=pl.BlockSpec(memory_space=pltpu.VMEM),
        scratch_shapes=[
            pltpu.VMEM((N_Y - 1, M_OUT, W), jnp.float32),
            pltpu.VMEM((N_XZ - 1, M_OUT, W), jnp.float32),
            pltpu.SemaphoreType.DMA((N_Y - 1,)),
            pltpu.SemaphoreType.DMA((N_Y - 1,)),
            pltpu.SemaphoreType.DMA((N_XZ - 1,)),
            pltpu.SemaphoreType.DMA((N_XZ - 1,)),
        ],
        compiler_params=pltpu.CompilerParams(
            collective_id=0, vmem_limit_bytes=63 * 1024 * 1024),
    )(q)


# device time: 211567 ns/iter; 3.2629x vs baseline; 1.3700x over previous
import functools

import jax
import jax.numpy as jnp
from jax import lax
from jax.experimental import pallas as pl
from jax.experimental.pallas import tpu as pltpu

N_Y = 4
N_XZ = 8
M_OUT = 512
N_COLS = 8192
W = N_COLS // N_XZ
HB = W // 2


def _ring_pos(my_x, my_z):
    return lax.select(my_x == 0, my_z, 7 - my_z)


def _pos_to_xz(pos):
    tx = pos // 4
    tz = lax.select(tx == 0, pos, 7 - pos)
    return tx, tz


def kernel(x, dy):
    my_x = lax.axis_index("x")
    my_z = lax.axis_index("z")
    pos = _ring_pos(my_x, my_z)
    c0 = pos * W
    dy_slice = lax.dynamic_slice_in_dim(dy, c0, W, axis=1)
    q = jnp.dot(x.T, dy_slice, preferred_element_type=jnp.float32)

    def body(q_ref, out_ref, comm_a, bcw, bccw, r_cw, r_ccw,
             a_send, a_recv, cw_send, cw_recv, ccw_send, ccw_recv):
        my_x = lax.axis_index("x")
        my_y = lax.axis_index("y")
        my_z = lax.axis_index("z")
        pos = _ring_pos(my_x, my_z)
        left = (my_y - 1) % N_Y
        right = (my_y + 1) % N_Y
        nxt = _pos_to_xz((pos + 1) % N_XZ)
        prv = _pos_to_xz((pos - 1) % N_XZ)
        nxt_id = (nxt[0], my_y, nxt[1])
        prv_id = (prv[0], my_y, prv[1])

        peers = [(my_x, left, my_z), (my_x, right, my_z), nxt_id, prv_id]
        barrier_sem = pltpu.get_barrier_semaphore()
        for p in peers:
            pl.semaphore_signal(
                barrier_sem, inc=1, device_id=p,
                device_id_type=pl.DeviceIdType.MESH)
        pl.semaphore_wait(barrier_sem, 4)

        for h in range(N_Y - 1):
            src = (q_ref.at[pl.ds(left * M_OUT, M_OUT), :] if h == 0
                   else comm_a.at[h - 1])
            rdma = pltpu.make_async_remote_copy(
                src_ref=src,
                dst_ref=comm_a.at[h],
                send_sem=a_send.at[h],
                recv_sem=a_recv.at[h],
                device_id=(my_x, right, my_z),
                device_id_type=pl.DeviceIdType.MESH)
            rdma.start()
            rdma.wait()
            d = (my_y - 2 - h) % N_Y
            comm_a[h] = comm_a[h] + q_ref[pl.ds(d * M_OUT, M_OUT), :]

        r_cw[...] = comm_a[N_Y - 2][:, 0:HB]
        r_ccw[...] = comm_a[N_Y - 2][:, HB:W]

        def cw_desc(h):
            return pltpu.make_async_remote_copy(
                src_ref=r_cw if h == 0 else bcw.at[h - 1],
                dst_ref=bcw.at[h],
                send_sem=cw_send.at[h],
                recv_sem=cw_recv.at[h],
                device_id=nxt_id,
                device_id_type=pl.DeviceIdType.MESH)

        def ccw_desc(h):
            return pltpu.make_async_remote_copy(
                src_ref=r_ccw if h == 0 else bccw.at[h - 1],
                dst_ref=bccw.at[h],
                send_sem=ccw_send.at[h],
                recv_sem=ccw_recv.at[h],
                device_id=prv_id,
                device_id_type=pl.DeviceIdType.MESH)

        cw_desc(0).start()
        ccw_desc(0).start()
        out_ref[:, pl.ds(pos * W, W)] = comm_a[N_Y - 2]

        for h in range(N_XZ - 1):
            cw_desc(h).wait_recv()
            ccw_desc(h).wait_recv()
            if h < N_XZ - 2:
                cw_desc(h + 1).start()
                ccw_desc(h + 1).start()
            o_cw = (pos - 1 - h) % N_XZ
            o_ccw = (pos + 1 + h) % N_XZ
            out_ref[:, pl.ds(o_cw * W, HB)] = bcw[h]
            out_ref[:, pl.ds(o_ccw * W + HB, HB)] = bccw[h]

        for h in range(N_XZ - 1):
            cw_desc(h).wait_send()
            ccw_desc(h).wait_send()

        @functools.partial(
            pl.run_scoped, second_barrier=pltpu.SemaphoreType.REGULAR)
        def _(second_barrier):
            for p in peers:
                pl.semaphore_signal(
                    second_barrier, inc=1, device_id=p,
                    device_id_type=pl.DeviceIdType.MESH)
            pl.semaphore_wait(second_barrier, 4)

    out_shape = jax.ShapeDtypeStruct((M_OUT, N_COLS), jnp.float32)
    return pl.pallas_call(
        body,
        out_shape=out_shape,
        in_specs=[pl.BlockSpec(memory_space=pltpu.VMEM)],
        out_specs=pl.BlockSpec(memory_space=pltpu.VMEM),
        scratch_shapes=[
            pltpu.VMEM((N_Y - 1, M_OUT, W), jnp.float32),
            pltpu.VMEM((N_XZ - 1, M_OUT, HB), jnp.float32),
            pltpu.VMEM((N_XZ - 1, M_OUT, HB), jnp.float32),
            pltpu.VMEM((M_OUT, HB), jnp.float32),
            pltpu.VMEM((M_OUT, HB), jnp.float32),
            pltpu.SemaphoreType.DMA((N_Y - 1,)),
            pltpu.SemaphoreType.DMA((N_Y - 1,)),
            pltpu.SemaphoreType.DMA((N_XZ - 1,)),
            pltpu.SemaphoreType.DMA((N_XZ - 1,)),
            pltpu.SemaphoreType.DMA((N_XZ - 1,)),
            pltpu.SemaphoreType.DMA((N_XZ - 1,)),
        ],
        compiler_params=pltpu.CompilerParams(
            collective_id=0, vmem_limit_bytes=63 * 1024 * 1024),
    )(q)
